# baseline (speedup 1.0000x reference)
import jax
import jax.numpy as jnp
from jax import lax
from jax.experimental import pallas as pl
from jax.experimental.pallas import tpu as pltpu
from jax.experimental.pallas import tpu_sc as plsc

BATCH = 16384
VOCAB = 1000000
DIM = 64

_NC = 2
_NS = 16
_NW = _NC * _NS
_BPW = BATCH // _NW
_L = 16
_K = 32
_NCHUNK = _BPW // _K
_KF = _K * DIM


def _body(feat_a_hbm, feat_b_hbm, wa_hbm, wb_hbm, out_hbm,
          idxa_v, idxb_v, fia_v, fib_v, bufa_v, bufb_v, rows_v, sem):
    wid = lax.axis_index("s") * _NC + lax.axis_index("c")
    base = wid * _BPW

    pltpu.sync_copy(feat_a_hbm.at[pl.ds(base, _BPW)], idxa_v)
    pltpu.sync_copy(feat_b_hbm.at[pl.ds(base, _BPW)], idxb_v)

    lane = lax.iota(jnp.int32, _L)
    cbase = [lane * VOCAB + cg * _L * VOCAB for cg in range(DIM // _L)]

    def chunk(c, _):
        k0 = c * _K
        for g in range(_K // _L):
            va = idxa_v[pl.ds(k0 + g * _L, _L)]
            vb = idxb_v[pl.ds(k0 + g * _L, _L)]
            for j in range(_L):
                slot = g * _L + j
                for cg in range(DIM // _L):
                    fia_v[pl.ds(slot * DIM + cg * _L, _L)] = cbase[cg] + va[j]
                    fib_v[pl.ds(slot * DIM + cg * _L, _L)] = cbase[cg] + vb[j]
        cpa = pltpu.async_copy(wa_hbm.at[fia_v], bufa_v, sem)
        cpb = pltpu.async_copy(wb_hbm.at[fib_v], bufb_v, sem)
        cpa.wait()
        cpb.wait()
        for slot in range(_K):
            for cg in range(DIM // _L):
                sl = pl.ds(slot * DIM + cg * _L, _L)
                rows_v[k0 + slot, 0, pl.ds(cg * _L, _L)] = bufa_v[sl]
                rows_v[k0 + slot, 1, pl.ds(cg * _L, _L)] = bufb_v[sl]
        return ()

    lax.fori_loop(0, _NCHUNK, chunk, ())

    pltpu.sync_copy(rows_v, out_hbm.at[pl.ds(base, _BPW)])


@jax.jit
def kernel(feat_a, feat_b, W_a, W_b):
    mesh = plsc.VectorSubcoreMesh(core_axis_name="c", subcore_axis_name="s")
    out = pl.kernel(
        _body,
        mesh=mesh,
        out_type=jax.ShapeDtypeStruct((BATCH, 2, DIM), jnp.float32),
        scratch_types=[
            pltpu.VMEM((_BPW,), jnp.int32),
            pltpu.VMEM((_BPW,), jnp.int32),
            pltpu.VMEM((_KF,), jnp.int32),
            pltpu.VMEM((_KF,), jnp.int32),
            pltpu.VMEM((_KF,), jnp.float32),
            pltpu.VMEM((_KF,), jnp.float32),
            pltpu.VMEM((_BPW, 2, DIM), jnp.float32),
            pltpu.SemaphoreType.DMA,
        ],
        compiler_params=pltpu.CompilerParams(use_tc_tiling_on_sc=False,
                                             needs_layout_passes=False),
    )(feat_a, feat_b,
      W_a.T.reshape(DIM * VOCAB), W_b.T.reshape(DIM * VOCAB))
    return out.reshape(BATCH, 2 * DIM)


# speedup vs baseline: 11.0944x; 11.0944x over previous
import jax
import jax.numpy as jnp
from jax import lax
from jax.experimental import pallas as pl
from jax.experimental.pallas import tpu as pltpu
from jax.experimental.pallas import tpu_sc as plsc

BATCH = 16384
VOCAB = 1000000
DIM = 64

_NC = 2
_NS = 16
_NW = _NC * _NS
_BPW = BATCH // _NW
_L = 16
_K = 32
_NCHUNK = _BPW // _K


def _body(feat_a_hbm, feat_b_hbm, w_hbm, out_hbm,
          idxa_v, idxb_v, rows_v, sem):
    wid = lax.axis_index("s") * _NC + lax.axis_index("c")
    base = wid * _BPW

    pltpu.sync_copy(feat_a_hbm.at[pl.ds(base, _BPW)], idxa_v)
    pltpu.sync_copy(feat_b_hbm.at[pl.ds(base, _BPW)], idxb_v)

    def chunk(c, _):
        k0 = c * _K
        copies = []
        for g in range(_K // _L):
            va = idxa_v[pl.ds(k0 + g * _L, _L)]
            vb = idxb_v[pl.ds(k0 + g * _L, _L)]
            for j in range(_L):
                slot = g * _L + j
                copies.append(pltpu.async_copy(
                    w_hbm.at[va[j], pl.ds(0, DIM)],
                    rows_v.at[k0 + slot, pl.ds(0, DIM)], sem))
                copies.append(pltpu.async_copy(
                    w_hbm.at[vb[j], pl.ds(DIM, DIM)],
                    rows_v.at[k0 + slot, pl.ds(DIM, DIM)], sem))
        for cp in copies:
            cp.wait()
        return ()

    lax.fori_loop(0, _NCHUNK, chunk, ())

    pltpu.sync_copy(rows_v, out_hbm.at[pl.ds(base, _BPW)])


@jax.jit
def kernel(feat_a, feat_b, W_a, W_b):
    w_ab = jnp.concatenate([W_a, W_b], axis=1)
    mesh = plsc.VectorSubcoreMesh(core_axis_name="c", subcore_axis_name="s")
    out = pl.kernel(
        _body,
        mesh=mesh,
        out_type=jax.ShapeDtypeStruct((BATCH, 2 * DIM), jnp.float32),
        scratch_types=[
            pltpu.VMEM((_BPW,), jnp.int32),
            pltpu.VMEM((_BPW,), jnp.int32),
            pltpu.VMEM((_BPW, 2 * DIM), jnp.float32),
            pltpu.SemaphoreType.DMA,
        ],
    )(feat_a, feat_b, w_ab)
    return out


# speedup vs baseline: 14.4926x; 1.3063x over previous
import jax
import jax.numpy as jnp
from jax import lax
from jax.experimental import pallas as pl
from jax.experimental.pallas import tpu as pltpu
from jax.experimental.pallas import tpu_sc as plsc

BATCH = 16384
VOCAB = 1000000
DIM = 64

_NC = 2
_NS = 16
_NW = _NC * _NS
_BPW = BATCH // _NW
_L = 16
_K = 32
_NCHUNK = _BPW // _K


def _body(feat_a_hbm, feat_b_hbm, wa_hbm, wb_hbm, out_hbm,
          idxa_v, idxb_v, rows_v, sem):
    wid = lax.axis_index("s") * _NC + lax.axis_index("c")
    base = wid * _BPW

    pltpu.sync_copy(feat_a_hbm.at[pl.ds(base, _BPW)], idxa_v)
    pltpu.sync_copy(feat_b_hbm.at[pl.ds(base, _BPW)], idxb_v)

    def chunk(c, _):
        k0 = c * _K
        copies = []
        for g in range(_K // _L):
            va = idxa_v[pl.ds(k0 + g * _L, _L)]
            vb = idxb_v[pl.ds(k0 + g * _L, _L)]
            for j in range(_L):
                slot = g * _L + j
                copies.append(pltpu.async_copy(
                    wa_hbm.at[va[j], :],
                    rows_v.at[k0 + slot, pl.ds(0, DIM)], sem))
                copies.append(pltpu.async_copy(
                    wb_hbm.at[vb[j], :],
                    rows_v.at[k0 + slot, pl.ds(DIM, DIM)], sem))
        for cp in copies:
            cp.wait()
        return ()

    lax.fori_loop(0, _NCHUNK, chunk, ())

    pltpu.sync_copy(rows_v, out_hbm.at[pl.ds(base, _BPW)])


@jax.jit
def kernel(feat_a, feat_b, W_a, W_b):
    mesh = plsc.VectorSubcoreMesh(core_axis_name="c", subcore_axis_name="s")
    out = pl.kernel(
        _body,
        mesh=mesh,
        out_type=jax.ShapeDtypeStruct((BATCH, 2 * DIM), jnp.float32),
        scratch_types=[
            pltpu.VMEM((_BPW,), jnp.int32),
            pltpu.VMEM((_BPW,), jnp.int32),
            pltpu.VMEM((_BPW, 2 * DIM), jnp.float32),
            pltpu.SemaphoreType.DMA,
        ],
    )(feat_a, feat_b, W_a, W_b)
    return out
